# baseline (speedup 1.0000x reference)
import functools

import numpy as np
import jax
import jax.numpy as jnp
from jax import lax
from jax.experimental import pallas as pl
from jax.experimental.pallas import tpu as pltpu
from jax.experimental.pallas import tpu_sc as plsc

_B, _L, _V, _D = 4096, 200, 100000, 64
_DP = 128
_NC, _NS, _LANES = 2, 16, 16
_NW = _NC * _NS
_BPW = _B // _NW
_LA, _LB = 104, 96
_RUNROLL = 8


def _pos_encoding():
    position = np.arange(_L, dtype=np.float32)[:, None]
    div_term = np.exp(
        np.arange(0, _D, 2, dtype=np.float32) * (-np.log(10000.0) / _D)
    ).astype(np.float32)
    pe = np.zeros((_L, _D), np.float32)
    pe[:, 0::2] = np.sin(position * div_term)
    pe[:, 1::2] = np.cos(position * div_term)
    return jnp.asarray(pe)


def _body(pos_hbm, tokens_hbm, table_hbm, out_hbm,
          pos_v, idx_v, rows0, rows1, o0, o1, gsem0, gsem1, osem0, osem1):
    wid = lax.axis_index("s") * _NC + lax.axis_index("c")
    base = wid * _BPW
    pltpu.sync_copy(tokens_hbm.at[pl.ds(base, _BPW)], idx_v)
    pltpu.sync_copy(pos_hbm, pos_v)

    rows = (rows0, rows1)
    outs = (o0, o1)
    gsem = (gsem0, gsem1)
    osem = (osem0, osem1)

    def fire_gather(i, s):
        pltpu.async_copy(
            table_hbm.at[idx_v.at[i, pl.ds(0, _LA)]],
            rows[s].at[pl.ds(0, _LA)], gsem[s]
        )
        pltpu.async_copy(
            table_hbm.at[idx_v.at[i, pl.ds(_LA, _LB)]],
            rows[s].at[pl.ds(_LA, _LB)], gsem[s]
        )

    def wait_gather(s):
        pltpu.make_async_copy(
            table_hbm.at[idx_v.at[0, pl.ds(0, _LA)]],
            rows[s].at[pl.ds(0, _LA)], gsem[s]
        ).wait()
        pltpu.make_async_copy(
            table_hbm.at[idx_v.at[0, pl.ds(_LA, _LB)]],
            rows[s].at[pl.ds(_LA, _LB)], gsem[s]
        ).wait()

    def fire_out(i, s):
        pltpu.async_copy(outs[s], out_hbm.at[base + i], osem[s])

    def wait_out(s):
        pltpu.make_async_copy(outs[s], out_hbm.at[base], osem[s]).wait()

    fire_gather(0, 0)

    def step(ko, carry):
        for s in range(2):
            i = 2 * ko + s
            n = 1 - s

            @pl.when(i < _BPW - 1)
            def _():
                fire_gather(i + 1, n)

            wait_gather(s)

            r_v = rows[s]
            w_v = outs[s]

            @pl.when(i > 1)
            def _():
                wait_out(s)

            def add_rows(ro, c):
                r0 = ro * _RUNROLL
                for rr in range(_RUNROLL):
                    for j in range(_D // _LANES):
                        sl = pl.ds(j * _LANES, _LANES)
                        w_v[r0 + rr, sl] = r_v[r0 + rr, sl] + pos_v[r0 + rr, sl]
                return c

            lax.fori_loop(0, _L // _RUNROLL, add_rows, 0)
            fire_out(i, s)
        return carry

    lax.fori_loop(0, _BPW // 2, step, 0)
    wait_out(0)
    wait_out(1)


@functools.partial(jax.jit, static_argnums=())
def kernel(tokens, table):
    pos = _pos_encoding()
    table_p = jnp.pad(table, ((0, 0), (0, _DP - _D)))
    mesh = plsc.VectorSubcoreMesh(core_axis_name="c", subcore_axis_name="s")
    run = pl.kernel(
        _body,
        out_type=jax.ShapeDtypeStruct((_B, _L, _D), jnp.float32),
        mesh=mesh,
        scratch_types=[
            pltpu.VMEM((_L, _D), jnp.float32),
            pltpu.VMEM((_BPW, _L), jnp.int32),
            pltpu.VMEM((_L, _DP), jnp.float32),
            pltpu.VMEM((_L, _DP), jnp.float32),
            pltpu.VMEM((_L, _D), jnp.float32),
            pltpu.VMEM((_L, _D), jnp.float32),
            pltpu.SemaphoreType.DMA,
            pltpu.SemaphoreType.DMA,
            pltpu.SemaphoreType.DMA,
            pltpu.SemaphoreType.DMA,
        ],
        compiler_params=pltpu.CompilerParams(use_tc_tiling_on_sc=False),
    )
    return run(pos, tokens, table_p)


# speedup vs baseline: 2.4502x; 2.4502x over previous
import functools

import numpy as np
import jax
import jax.numpy as jnp
from jax import lax
from jax.experimental import pallas as pl
from jax.experimental.pallas import tpu as pltpu
from jax.experimental.pallas import tpu_sc as plsc

_B, _L, _V, _D = 4096, 200, 100000, 64
_DP = 128
_NC, _NS, _LANES = 2, 16, 16
_NW = _NC * _NS
_BPW = _B // _NW
_LA, _LB = 104, 96
_RUNROLL = 8


def _pos_encoding():
    position = np.arange(_L, dtype=np.float32)[:, None]
    div_term = np.exp(
        np.arange(0, _D, 2, dtype=np.float32) * (-np.log(10000.0) / _D)
    ).astype(np.float32)
    pe = np.zeros((_L, _D), np.float32)
    pe[:, 0::2] = np.sin(position * div_term)
    pe[:, 1::2] = np.cos(position * div_term)
    return jnp.asarray(pe)


def _body(pos_hbm, tokens_hbm, table_hbm, out_hbm,
          pos_v, idx_v, rows0, rows1, o0, o1, gsem0, gsem1, osem0, osem1):
    wid = lax.axis_index("s") * _NC + lax.axis_index("c")
    base = wid * _BPW
    pltpu.sync_copy(tokens_hbm.at[pl.ds(base, _BPW)], idx_v)
    pltpu.sync_copy(pos_hbm, pos_v)

    rows = (rows0, rows1)
    outs = (o0, o1)
    gsem = (gsem0, gsem1)
    osem = (osem0, osem1)

    def fire_gather(i, s):
        pltpu.async_copy(
            table_hbm.at[idx_v.at[i, pl.ds(0, _LA)]],
            rows[s].at[pl.ds(0, _LA)], gsem[s]
        )
        pltpu.async_copy(
            table_hbm.at[idx_v.at[i, pl.ds(_LA, _LB)]],
            rows[s].at[pl.ds(_LA, _LB)], gsem[s]
        )

    def wait_gather(s):
        pltpu.make_async_copy(
            table_hbm.at[idx_v.at[0, pl.ds(0, _LA)]],
            rows[s].at[pl.ds(0, _LA)], gsem[s]
        ).wait()
        pltpu.make_async_copy(
            table_hbm.at[idx_v.at[0, pl.ds(_LA, _LB)]],
            rows[s].at[pl.ds(_LA, _LB)], gsem[s]
        ).wait()

    def fire_out(i, s):
        pltpu.async_copy(
            outs[s],
            out_hbm.at[pl.ds((base + i) * _L, _L), pl.ds(0, _D)],
            osem[s],
        )

    def wait_out(s):
        pltpu.make_async_copy(
            outs[s],
            out_hbm.at[pl.ds(base * _L, _L), pl.ds(0, _D)],
            osem[s],
        ).wait()

    fire_gather(0, 0)

    def step(ko, carry):
        for s in range(2):
            i = 2 * ko + s
            n = 1 - s

            @pl.when(i < _BPW - 1)
            def _():
                fire_gather(i + 1, n)

            wait_gather(s)

            r_v = rows[s]
            w_v = outs[s]

            @pl.when(i > 1)
            def _():
                wait_out(s)

            def add_rows(ro, c):
                r0 = ro * _RUNROLL
                for rr in range(_RUNROLL):
                    for j in range(_D // _LANES):
                        sl = pl.ds(j * _LANES, _LANES)
                        w_v[r0 + rr, sl] = r_v[r0 + rr, sl] + pos_v[r0 + rr, sl]
                return c

            lax.fori_loop(0, _L // _RUNROLL, add_rows, 0)
            fire_out(i, s)
        return carry

    lax.fori_loop(0, _BPW // 2, step, 0)
    wait_out(0)
    wait_out(1)


@functools.partial(jax.jit, static_argnums=())
def kernel(tokens, table):
    pos = _pos_encoding()
    mesh = plsc.VectorSubcoreMesh(core_axis_name="c", subcore_axis_name="s")
    run = pl.kernel(
        _body,
        out_type=jax.ShapeDtypeStruct((_B * _L, _DP), jnp.float32),
        mesh=mesh,
        scratch_types=[
            pltpu.VMEM((_L, _D), jnp.float32),
            pltpu.VMEM((_BPW, _L), jnp.int32),
            pltpu.VMEM((_L, _D), jnp.float32),
            pltpu.VMEM((_L, _D), jnp.float32),
            pltpu.VMEM((_L, _D), jnp.float32),
            pltpu.VMEM((_L, _D), jnp.float32),
            pltpu.SemaphoreType.DMA,
            pltpu.SemaphoreType.DMA,
            pltpu.SemaphoreType.DMA,
            pltpu.SemaphoreType.DMA,
        ],
        compiler_params=pltpu.CompilerParams(use_tc_tiling_on_sc=False),
    )
    out_p = run(pos, tokens, table)
    return out_p[:, :_D].reshape(_B, _L, _D)
